# Initial kernel scaffold; baseline (speedup 1.0000x reference)
import functools

import jax
import jax.numpy as jnp
from jax import lax
from jax.experimental import pallas as pl
from jax.experimental.pallas import tpu as pltpu
from jax.experimental.pallas import tpu_sc as plsc

N_NODES = 10000
N_PAD = 10240
N_EDGES = 320000
D = 128

NC = 2
NS = 16
NW = NC * NS
EPT = N_EDGES // NW
CH = 80
NCH = EPT // CH
ROWS_PER_TILE = N_PAD // NS


def _pre_body(x_ref, w1a_ref, w1b_ref, b1_ref, xa_ref, xb_ref):
    x = x_ref[...]
    xa_ref[...] = (
        jnp.dot(x, w1a_ref[...], preferred_element_type=jnp.float32)
        + b1_ref[...]
    )
    xb_ref[...] = jnp.dot(x, w1b_ref[...], preferred_element_type=jnp.float32)


def _precompute(x, w1a, w1b, b1):
    return pl.pallas_call(
        _pre_body,
        out_shape=[
            jax.ShapeDtypeStruct((N_NODES, D), jnp.float32),
            jax.ShapeDtypeStruct((N_NODES, D), jnp.float32),
        ],
    )(x, w1a, w1b, b1)


def _gather_body(row2d, col2d, xa, xb, ga, gb, ir, ic, bufa, bufb, sema, semb):
    wid = lax.axis_index("s") * NC + lax.axis_index("c")
    base = wid * EPT
    pltpu.sync_copy(row2d.at[pl.ds(wid * NCH, NCH)], ir)
    pltpu.sync_copy(col2d.at[pl.ds(wid * NCH, NCH)], ic)

    def chunk(j, carry):
        cpa = pltpu.async_copy(xa.at[ir.at[j]], bufa, sema)
        cpb = pltpu.async_copy(xb.at[ic.at[j]], bufb, semb)
        cpa.wait()
        cpb.wait()
        pltpu.sync_copy(bufa, ga.at[pl.ds(base + j * CH, CH)])
        pltpu.sync_copy(bufb, gb.at[pl.ds(base + j * CH, CH)])
        return carry

    lax.fori_loop(0, NCH, chunk, 0)


def _gather(row2d, col2d, xa, xb):
    mesh = plsc.VectorSubcoreMesh(core_axis_name="c", subcore_axis_name="s")
    f = pl.kernel(
        _gather_body,
        out_type=[
            jax.ShapeDtypeStruct((N_EDGES, D), jnp.float32),
            jax.ShapeDtypeStruct((N_EDGES, D), jnp.float32),
        ],
        mesh=mesh,
        scratch_types=[
            pltpu.VMEM((NCH, CH), jnp.int32),
            pltpu.VMEM((NCH, CH), jnp.int32),
            pltpu.VMEM((CH, D), jnp.float32),
            pltpu.VMEM((CH, D), jnp.float32),
            pltpu.SemaphoreType.DMA,
            pltpu.SemaphoreType.DMA,
        ],
    )
    return f(row2d, col2d, xa, xb)


BE = 1280


def _mlp_body(ga_ref, gb_ref, w2_ref, b2_ref, h_ref):
    g = jax.nn.softplus(ga_ref[...] + gb_ref[...])
    z = jnp.dot(g, w2_ref[...], preferred_element_type=jnp.float32) + b2_ref[...]
    h_ref[...] = jax.nn.softplus(z)


def _mlp(ga, gb, w2, b2):
    grid = N_EDGES // BE
    return pl.pallas_call(
        _mlp_body,
        grid=(grid,),
        in_specs=[
            pl.BlockSpec((BE, D), lambda i: (i, 0)),
            pl.BlockSpec((BE, D), lambda i: (i, 0)),
            pl.BlockSpec((D, D), lambda i: (0, 0)),
            pl.BlockSpec((1, D), lambda i: (0, 0)),
        ],
        out_specs=pl.BlockSpec((BE, D), lambda i: (i, 0)),
        out_shape=jax.ShapeDtypeStruct((N_EDGES, D), jnp.float32),
    )(ga, gb, w2, b2)


def _scatter_body(row2d, h, parts, cnts, ir, hbuf, ones, zbuf, zbuf2, acc, acc_c):
    c = lax.axis_index("c")
    s = lax.axis_index("s")
    wid = s * NC + c
    base = wid * EPT

    zv = jnp.zeros((16,), jnp.float32)
    ov = jnp.ones((16,), jnp.float32)

    def fill_ones(r, carry):
        ones[r, :] = ov
        return carry

    lax.fori_loop(0, CH, fill_ones, 0)

    def fill_z(r, carry):
        for k in range(D // 16):
            zbuf[r, pl.ds(k * 16, 16)] = zv
        zbuf2[r, :] = zv
        return carry

    lax.fori_loop(0, ROWS_PER_TILE, fill_z, 0)

    pltpu.sync_copy(zbuf, acc.at[pl.ds(s * ROWS_PER_TILE, ROWS_PER_TILE)])
    pltpu.sync_copy(zbuf2, acc_c.at[pl.ds(s * ROWS_PER_TILE, ROWS_PER_TILE)])
    plsc.subcore_barrier()

    pltpu.sync_copy(row2d.at[pl.ds(wid * NCH, NCH)], ir)

    def chunk(j, carry):
        pltpu.sync_copy(h.at[pl.ds(base + j * CH, CH)], hbuf)
        pltpu.sync_copy(hbuf, acc.at[ir.at[j]], add=True)
        pltpu.sync_copy(ones, acc_c.at[ir.at[j]], add=True)
        return carry

    lax.fori_loop(0, NCH, chunk, 0)
    plsc.subcore_barrier()

    pltpu.sync_copy(
        acc.at[pl.ds(s * ROWS_PER_TILE, ROWS_PER_TILE)],
        parts.at[c].at[pl.ds(s * ROWS_PER_TILE, ROWS_PER_TILE)],
    )
    pltpu.sync_copy(
        acc_c.at[pl.ds(s * ROWS_PER_TILE, ROWS_PER_TILE)],
        cnts.at[c].at[pl.ds(s * ROWS_PER_TILE, ROWS_PER_TILE)],
    )


def _scatter(row2d, h):
    mesh = plsc.VectorSubcoreMesh(core_axis_name="c", subcore_axis_name="s")
    f = pl.kernel(
        _scatter_body,
        out_type=[
            jax.ShapeDtypeStruct((NC, N_PAD, D), jnp.float32),
            jax.ShapeDtypeStruct((NC, N_PAD, 16), jnp.float32),
        ],
        mesh=mesh,
        scratch_types=[
            pltpu.VMEM((NCH, CH), jnp.int32),
            pltpu.VMEM((CH, D), jnp.float32),
            pltpu.VMEM((CH, 16), jnp.float32),
            pltpu.VMEM((ROWS_PER_TILE, D), jnp.float32),
            pltpu.VMEM((ROWS_PER_TILE, 16), jnp.float32),
            pltpu.VMEM_SHARED((N_PAD, D), jnp.float32),
            pltpu.VMEM_SHARED((N_PAD, 16), jnp.float32),
        ],
    )
    return f(row2d, h)


def _final_body(parts_ref, cnts_ref, w3_ref, b3_ref, out_ref):
    p = parts_ref[0, :N_NODES, :] + parts_ref[1, :N_NODES, :]
    cnt = cnts_ref[0, :N_NODES, 0:1] + cnts_ref[1, :N_NODES, 0:1]
    out_ref[...] = (
        jnp.dot(p, w3_ref[...], preferred_element_type=jnp.float32)
        + cnt * b3_ref[...]
    )


def _final(parts, cnts, w3, b3):
    return pl.pallas_call(
        _final_body,
        out_shape=jax.ShapeDtypeStruct((N_NODES, D), jnp.float32),
    )(parts, cnts, w3, b3)


def kernel(x, edge_idx, W1, b1, W2, b2, W3, b3):
    row = edge_idx[0].astype(jnp.int32)
    col = edge_idx[1].astype(jnp.int32)
    row2d = row.reshape(NW * NCH, CH)
    col2d = col.reshape(NW * NCH, CH)
    xa, xb = _precompute(x, W1[:D], W1[D:], b1.reshape(1, D))
    ga, gb = _gather(row2d, col2d, xa, xb)
    h = _mlp(ga, gb, W2, b2.reshape(1, D))
    parts, cnts = _scatter(row2d, h)
    return _final(parts, cnts, W3, b3.reshape(1, D))


# speedup vs baseline: 2.9782x; 2.9782x over previous
import functools

import jax
import jax.numpy as jnp
from jax import lax
from jax.experimental import pallas as pl
from jax.experimental.pallas import tpu as pltpu
from jax.experimental.pallas import tpu_sc as plsc

N_NODES = 10000
N_PAD = 10240
N_EDGES = 320000
D = 128

NC = 2
NS = 16
NW = NC * NS
EPT = N_EDGES // NW
CH = 80
NCH = EPT // CH
ROWS_PER_TILE = N_PAD // NS


def _pre_body(x_ref, w1a_ref, w1b_ref, b1_ref, xa_ref, xb_ref):
    x = x_ref[...]
    xa_ref[...] = (
        jnp.dot(x, w1a_ref[...], preferred_element_type=jnp.float32)
        + b1_ref[...]
    )
    xb_ref[...] = jnp.dot(x, w1b_ref[...], preferred_element_type=jnp.float32)


def _precompute(x, w1a, w1b, b1):
    return pl.pallas_call(
        _pre_body,
        out_shape=[
            jax.ShapeDtypeStruct((N_NODES, D), jnp.float32),
            jax.ShapeDtypeStruct((N_NODES, D), jnp.float32),
        ],
    )(x, w1a, w1b, b1)


def _gather_body(row3d, col3d, xa, xb, ga, gb, ir, ic, bufa, bufb, sema, semb):
    wid = lax.axis_index("s") * NC + lax.axis_index("c")
    base = wid * EPT
    pltpu.sync_copy(row3d.at[wid], ir)
    pltpu.sync_copy(col3d.at[wid], ic)

    def chunk(j, carry):
        cpa = pltpu.async_copy(xa.at[ir.at[j]], bufa, sema)
        cpb = pltpu.async_copy(xb.at[ic.at[j]], bufb, semb)
        cpa.wait()
        cpb.wait()
        pltpu.sync_copy(bufa, ga.at[pl.ds(base + j * CH, CH)])
        pltpu.sync_copy(bufb, gb.at[pl.ds(base + j * CH, CH)])
        return carry

    lax.fori_loop(0, NCH, chunk, 0)


def _gather(row3d, col3d, xa, xb):
    mesh = plsc.VectorSubcoreMesh(core_axis_name="c", subcore_axis_name="s")
    f = pl.kernel(
        _gather_body,
        out_type=[
            jax.ShapeDtypeStruct((N_EDGES, D), jnp.float32),
            jax.ShapeDtypeStruct((N_EDGES, D), jnp.float32),
        ],
        mesh=mesh,
        scratch_types=[
            pltpu.VMEM((NCH, CH), jnp.int32),
            pltpu.VMEM((NCH, CH), jnp.int32),
            pltpu.VMEM((CH, D), jnp.float32),
            pltpu.VMEM((CH, D), jnp.float32),
            pltpu.SemaphoreType.DMA,
            pltpu.SemaphoreType.DMA,
        ],
        compiler_params=pltpu.CompilerParams(use_tc_tiling_on_sc=False),
    )
    return f(row3d, col3d, xa, xb)


BE = 1280


def _mlp_body(ga_ref, gb_ref, w2_ref, b2_ref, h_ref):
    g = jax.nn.softplus(ga_ref[...] + gb_ref[...])
    z = jnp.dot(g, w2_ref[...], preferred_element_type=jnp.float32) + b2_ref[...]
    h_ref[...] = jax.nn.softplus(z)


def _mlp(ga, gb, w2, b2):
    grid = N_EDGES // BE
    return pl.pallas_call(
        _mlp_body,
        grid=(grid,),
        in_specs=[
            pl.BlockSpec((BE, D), lambda i: (i, 0)),
            pl.BlockSpec((BE, D), lambda i: (i, 0)),
            pl.BlockSpec((D, D), lambda i: (0, 0)),
            pl.BlockSpec((1, D), lambda i: (0, 0)),
        ],
        out_specs=pl.BlockSpec((BE, D), lambda i: (i, 0)),
        out_shape=jax.ShapeDtypeStruct((N_EDGES, D), jnp.float32),
    )(ga, gb, w2, b2)


def _scatter_body(row3d, h, parts, cnts, ir, hbuf, ones, acc, acc_c):
    c = lax.axis_index("c")
    s = lax.axis_index("s")
    wid = s * NC + c
    base = wid * EPT

    zv = jnp.zeros((16,), jnp.float32)
    ov = jnp.ones((16,), jnp.float32)

    def fill_z(r, carry):
        for k in range(D // 16):
            hbuf[r, pl.ds(k * 16, 16)] = zv
        ones[r, :] = zv
        return carry

    lax.fori_loop(0, CH, fill_z, 0)

    def zero_stripe(t, carry):
        off = s * ROWS_PER_TILE + t * CH
        pltpu.sync_copy(hbuf, acc.at[pl.ds(off, CH)])
        pltpu.sync_copy(ones, acc_c.at[pl.ds(off, CH)])
        return carry

    lax.fori_loop(0, ROWS_PER_TILE // CH, zero_stripe, 0)

    def fill_ones(r, carry):
        ones[r, :] = ov
        return carry

    lax.fori_loop(0, CH, fill_ones, 0)
    plsc.subcore_barrier()

    pltpu.sync_copy(row3d.at[wid], ir)

    def chunk(j, carry):
        pltpu.sync_copy(h.at[pl.ds(base + j * CH, CH)], hbuf)
        pltpu.sync_copy(hbuf, acc.at[ir.at[j]], add=True)
        pltpu.sync_copy(ones, acc_c.at[ir.at[j]], add=True)
        return carry

    lax.fori_loop(0, NCH, chunk, 0)
    plsc.subcore_barrier()

    pltpu.sync_copy(
        acc.at[pl.ds(s * ROWS_PER_TILE, ROWS_PER_TILE)],
        parts.at[c].at[pl.ds(s * ROWS_PER_TILE, ROWS_PER_TILE)],
    )
    pltpu.sync_copy(
        acc_c.at[pl.ds(s * ROWS_PER_TILE, ROWS_PER_TILE)],
        cnts.at[c].at[pl.ds(s * ROWS_PER_TILE, ROWS_PER_TILE)],
    )


def _scatter(row3d, h):
    mesh = plsc.VectorSubcoreMesh(core_axis_name="c", subcore_axis_name="s")
    f = pl.kernel(
        _scatter_body,
        out_type=[
            jax.ShapeDtypeStruct((NC, N_PAD, D), jnp.float32),
            jax.ShapeDtypeStruct((NC, N_PAD, 16), jnp.float32),
        ],
        mesh=mesh,
        scratch_types=[
            pltpu.VMEM((NCH, CH), jnp.int32),
            pltpu.VMEM((CH, D), jnp.float32),
            pltpu.VMEM((CH, 16), jnp.float32),
            pltpu.VMEM_SHARED((N_PAD, D), jnp.float32),
            pltpu.VMEM_SHARED((N_PAD, 16), jnp.float32),
        ],
        compiler_params=pltpu.CompilerParams(use_tc_tiling_on_sc=False),
    )
    return f(row3d, h)


def _final_body(parts_ref, cnts_ref, w3_ref, b3_ref, out_ref):
    p = parts_ref[0, :N_NODES, :] + parts_ref[1, :N_NODES, :]
    cnt = cnts_ref[0, :N_NODES, 0:1] + cnts_ref[1, :N_NODES, 0:1]
    out_ref[...] = (
        jnp.dot(p, w3_ref[...], preferred_element_type=jnp.float32)
        + cnt * b3_ref[...]
    )


def _final(parts, cnts, w3, b3):
    return pl.pallas_call(
        _final_body,
        out_shape=jax.ShapeDtypeStruct((N_NODES, D), jnp.float32),
    )(parts, cnts, w3, b3)


def kernel(x, edge_idx, W1, b1, W2, b2, W3, b3):
    row = edge_idx[0].astype(jnp.int32)
    col = edge_idx[1].astype(jnp.int32)
    row3d = row.reshape(NW, NCH, CH)
    col3d = col.reshape(NW, NCH, CH)
    xa, xb = _precompute(x, W1[:D], W1[D:], b1.reshape(1, D))
    ga, gb = _gather(row3d, col3d, xa, xb)
    h = _mlp(ga, gb, W2, b2.reshape(1, D))
    parts, cnts = _scatter(row3d, h)
    return _final(parts, cnts, W3, b3.reshape(1, D))


# speedup vs baseline: 4.2024x; 1.4110x over previous
import functools

import jax
import jax.numpy as jnp
from jax import lax
from jax.experimental import pallas as pl
from jax.experimental.pallas import tpu as pltpu
from jax.experimental.pallas import tpu_sc as plsc

N_NODES = 10000
N_PAD = 10240
N_EDGES = 320000
D = 128

NC = 2
NS = 16
NW = NC * NS
EPT = N_EDGES // NW
CH = 80
NCH = EPT // CH
NPAIR = (NCH - 1) // 2
ROWS_PER_TILE = N_PAD // NS


def _pre_body(x_ref, w1a_ref, w1b_ref, b1_ref, xa_ref, xb_ref):
    x = x_ref[...]
    xa_ref[...] = (
        jnp.dot(x, w1a_ref[...], preferred_element_type=jnp.float32)
        + b1_ref[...]
    )
    xb_ref[...] = jnp.dot(x, w1b_ref[...], preferred_element_type=jnp.float32)


def _precompute(x, w1a, w1b, b1):
    return pl.pallas_call(
        _pre_body,
        out_shape=[
            jax.ShapeDtypeStruct((N_NODES, D), jnp.float32),
            jax.ShapeDtypeStruct((N_NODES, D), jnp.float32),
        ],
    )(x, w1a, w1b, b1)


def _gather_body(row3d, col3d, xa, xb, g, cnts,
                 ir, ic, ba0, bb0, ba1, bb1, ones, acc_c,
                 sa0, sb0, sa1, sb1):
    c = lax.axis_index("c")
    s = lax.axis_index("s")
    wid = s * NC + c
    base = wid * EPT
    pltpu.sync_copy(row3d.at[wid], ir)
    pltpu.sync_copy(col3d.at[wid], ic)

    zv = jnp.zeros((16,), jnp.float32)
    ov = jnp.ones((16,), jnp.float32)

    def fill_z(r, carry):
        ones[r, :] = zv
        return carry

    lax.fori_loop(0, CH, fill_z, 0)

    def zero_stripe(t, carry):
        pltpu.sync_copy(ones, acc_c.at[pl.ds(s * ROWS_PER_TILE + t * CH, CH)])
        return carry

    lax.fori_loop(0, ROWS_PER_TILE // CH, zero_stripe, 0)

    def fill_ones(r, carry):
        ones[r, :] = ov
        return carry

    lax.fori_loop(0, CH, fill_ones, 0)
    plsc.subcore_barrier()

    def start(j, ba, bb, sa, sb):
        pltpu.async_copy(xa.at[ir.at[j]], ba, sa)
        pltpu.async_copy(xb.at[ic.at[j]], bb, sb)

    def wait(ba, bb, sa, sb):
        pltpu.make_async_copy(xa.at[ir.at[0]], ba, sa).wait()
        pltpu.make_async_copy(xb.at[ic.at[0]], bb, sb).wait()

    def finish(j, ba, bb):
        def addrow(r, carry):
            for k in range(D // 16):
                sl = pl.ds(k * 16, 16)
                plsc.addupdate(ba.at[r, sl], bb[r, sl])
            return carry

        lax.fori_loop(0, CH, addrow, 0)
        pltpu.sync_copy(ba, g.at[pl.ds(base + j * CH, CH)])
        pltpu.sync_copy(ones, acc_c.at[ir.at[j]], add=True)

    start(0, ba0, bb0, sa0, sb0)

    def pair(jj, carry):
        j0 = 2 * jj
        start(j0 + 1, ba1, bb1, sa1, sb1)
        wait(ba0, bb0, sa0, sb0)
        finish(j0, ba0, bb0)
        start(j0 + 2, ba0, bb0, sa0, sb0)
        wait(ba1, bb1, sa1, sb1)
        finish(j0 + 1, ba1, bb1)
        return carry

    lax.fori_loop(0, NPAIR, pair, 0)
    wait(ba0, bb0, sa0, sb0)
    finish(NCH - 1, ba0, bb0)
    plsc.subcore_barrier()

    pltpu.sync_copy(
        acc_c.at[pl.ds(s * ROWS_PER_TILE, ROWS_PER_TILE)],
        cnts.at[c].at[pl.ds(s * ROWS_PER_TILE, ROWS_PER_TILE)],
    )


def _gather(row3d, col3d, xa, xb):
    mesh = plsc.VectorSubcoreMesh(core_axis_name="c", subcore_axis_name="s")
    f = pl.kernel(
        _gather_body,
        out_type=[
            jax.ShapeDtypeStruct((N_EDGES, D), jnp.float32),
            jax.ShapeDtypeStruct((NC, N_PAD, 16), jnp.float32),
        ],
        mesh=mesh,
        scratch_types=[
            pltpu.VMEM((NCH, CH), jnp.int32),
            pltpu.VMEM((NCH, CH), jnp.int32),
            pltpu.VMEM((CH, D), jnp.float32),
            pltpu.VMEM((CH, D), jnp.float32),
            pltpu.VMEM((CH, D), jnp.float32),
            pltpu.VMEM((CH, D), jnp.float32),
            pltpu.VMEM((CH, 16), jnp.float32),
            pltpu.VMEM_SHARED((N_PAD, 16), jnp.float32),
            pltpu.SemaphoreType.DMA,
            pltpu.SemaphoreType.DMA,
            pltpu.SemaphoreType.DMA,
            pltpu.SemaphoreType.DMA,
        ],
        compiler_params=pltpu.CompilerParams(use_tc_tiling_on_sc=False),
    )
    return f(row3d, col3d, xa, xb)


BE = 1280


def _mlp_body(g_ref, w2_ref, b2_ref, h_ref):
    g = jax.nn.softplus(g_ref[...])
    z = jnp.dot(g, w2_ref[...], preferred_element_type=jnp.float32) + b2_ref[...]
    h_ref[...] = jax.nn.softplus(z)


def _mlp(g, w2, b2):
    grid = N_EDGES // BE
    return pl.pallas_call(
        _mlp_body,
        grid=(grid,),
        in_specs=[
            pl.BlockSpec((BE, D), lambda i: (i, 0)),
            pl.BlockSpec((D, D), lambda i: (0, 0)),
            pl.BlockSpec((1, D), lambda i: (0, 0)),
        ],
        out_specs=pl.BlockSpec((BE, D), lambda i: (i, 0)),
        out_shape=jax.ShapeDtypeStruct((N_EDGES, D), jnp.float32),
    )(g, w2, b2)


def _scatter_body(row3d, h, parts, ir, hb0, hb1, zb, acc, sh0, sh1):
    c = lax.axis_index("c")
    s = lax.axis_index("s")
    wid = s * NC + c
    base = wid * EPT

    zv = jnp.zeros((16,), jnp.float32)

    def fill_z(r, carry):
        for k in range(D // 16):
            zb[r, pl.ds(k * 16, 16)] = zv
        return carry

    lax.fori_loop(0, CH, fill_z, 0)

    def zero_stripe(t, carry):
        pltpu.sync_copy(zb, acc.at[pl.ds(s * ROWS_PER_TILE + t * CH, CH)])
        return carry

    lax.fori_loop(0, ROWS_PER_TILE // CH, zero_stripe, 0)
    pltpu.sync_copy(row3d.at[wid], ir)
    plsc.subcore_barrier()

    def start(j, hb, sh):
        pltpu.async_copy(h.at[pl.ds(base + j * CH, CH)], hb, sh)

    def wait(hb, sh):
        pltpu.make_async_copy(h.at[pl.ds(0, CH)], hb, sh).wait()

    def finish(j, hb):
        pltpu.sync_copy(hb, acc.at[ir.at[j]], add=True)

    start(0, hb0, sh0)

    def pair(jj, carry):
        j0 = 2 * jj
        start(j0 + 1, hb1, sh1)
        wait(hb0, sh0)
        finish(j0, hb0)
        start(j0 + 2, hb0, sh0)
        wait(hb1, sh1)
        finish(j0 + 1, hb1)
        return carry

    lax.fori_loop(0, NPAIR, pair, 0)
    wait(hb0, sh0)
    finish(NCH - 1, hb0)
    plsc.subcore_barrier()

    pltpu.sync_copy(
        acc.at[pl.ds(s * ROWS_PER_TILE, ROWS_PER_TILE)],
        parts.at[c].at[pl.ds(s * ROWS_PER_TILE, ROWS_PER_TILE)],
    )


def _scatter(row3d, h):
    mesh = plsc.VectorSubcoreMesh(core_axis_name="c", subcore_axis_name="s")
    f = pl.kernel(
        _scatter_body,
        out_type=[
            jax.ShapeDtypeStruct((NC, N_PAD, D), jnp.float32),
        ],
        mesh=mesh,
        scratch_types=[
            pltpu.VMEM((NCH, CH), jnp.int32),
            pltpu.VMEM((CH, D), jnp.float32),
            pltpu.VMEM((CH, D), jnp.float32),
            pltpu.VMEM((CH, D), jnp.float32),
            pltpu.VMEM_SHARED((N_PAD, D), jnp.float32),
            pltpu.SemaphoreType.DMA,
            pltpu.SemaphoreType.DMA,
        ],
        compiler_params=pltpu.CompilerParams(use_tc_tiling_on_sc=False),
    )
    return f(row3d, h)


def _final_body(parts_ref, cnts_ref, w3_ref, b3_ref, out_ref):
    p = parts_ref[0, :N_NODES, :] + parts_ref[1, :N_NODES, :]
    cnt = cnts_ref[0, :N_NODES, 0:1] + cnts_ref[1, :N_NODES, 0:1]
    out_ref[...] = (
        jnp.dot(p, w3_ref[...], preferred_element_type=jnp.float32)
        + cnt * b3_ref[...]
    )


def _final(parts, cnts, w3, b3):
    return pl.pallas_call(
        _final_body,
        out_shape=jax.ShapeDtypeStruct((N_NODES, D), jnp.float32),
    )(parts, cnts, w3, b3)


def kernel(x, edge_idx, W1, b1, W2, b2, W3, b3):
    row = edge_idx[0].astype(jnp.int32)
    col = edge_idx[1].astype(jnp.int32)
    row3d = row.reshape(NW, NCH, CH)
    col3d = col.reshape(NW, NCH, CH)
    xa, xb = _precompute(x, W1[:D], W1[D:], b1.reshape(1, D))
    g, cnts = _gather(row3d, col3d, xa, xb)
    h = _mlp(g, W2, b2.reshape(1, D))
    parts = _scatter(row3d, h)
    return _final(parts[0], cnts, W3, b3.reshape(1, D))


# speedup vs baseline: 4.8841x; 1.1622x over previous
import functools

import jax
import jax.numpy as jnp
from jax import lax
from jax.experimental import pallas as pl
from jax.experimental.pallas import tpu as pltpu
from jax.experimental.pallas import tpu_sc as plsc

N_NODES = 10000
N_PAD = 10240
N_EDGES = 320000
D = 128

NC = 2
NS = 16
NW = NC * NS
CH = 80
ROWS_PER_TILE = N_PAD // NS

NB = 5
EB = N_EDGES // NB


def _pipelined_chunks(nch, start, wait0, wait1, finish):
    start(0, 0)
    npair = (nch - 1) // 2 if nch % 2 else nch // 2 - 1

    def pair(jj, carry):
        j0 = 2 * jj
        start(j0 + 1, 1)
        wait0()
        finish(j0, 0)
        start(j0 + 2, 0)
        wait1()
        finish(j0 + 1, 1)
        return carry

    lax.fori_loop(0, npair, pair, 0)
    if nch % 2:
        wait0()
        finish(nch - 1, 0)
    else:
        j = 2 * npair
        start(j + 1, 1)
        wait0()
        finish(j, 0)
        wait1()
        finish(j + 1, 1)


def _pre_body(x_ref, w1a_ref, w1b_ref, b1_ref, xa_ref, xb_ref):
    x = x_ref[...]
    xa_ref[...] = (
        jnp.dot(x, w1a_ref[...], preferred_element_type=jnp.float32)
        + b1_ref[...]
    )
    xb_ref[...] = jnp.dot(x, w1b_ref[...], preferred_element_type=jnp.float32)


def _precompute(x, w1a, w1b, b1):
    return pl.pallas_call(
        _pre_body,
        out_shape=[
            jax.ShapeDtypeStruct((N_NODES, D), jnp.float32),
            jax.ShapeDtypeStruct((N_NODES, D), jnp.float32),
        ],
    )(x, w1a, w1b, b1)


def _gather_body(ept, nch, row3d, col3d, xa, xb, g,
                 ir, ic, ba0, bb0, ba1, bb1, sa0, sb0, sa1, sb1):
    wid = lax.axis_index("s") * NC + lax.axis_index("c")
    base = wid * ept
    pltpu.sync_copy(row3d.at[wid], ir)
    pltpu.sync_copy(col3d.at[wid], ic)
    bufs = ((ba0, bb0, sa0, sb0), (ba1, bb1, sa1, sb1))

    def start(j, slot):
        ba, bb, sa, sb = bufs[slot]
        pltpu.async_copy(xa.at[ir.at[j]], ba, sa)
        pltpu.async_copy(xb.at[ic.at[j]], bb, sb)

    def wait(slot):
        ba, bb, sa, sb = bufs[slot]
        pltpu.make_async_copy(xa.at[ir.at[0]], ba, sa).wait()
        pltpu.make_async_copy(xb.at[ic.at[0]], bb, sb).wait()

    def finish(j, slot):
        ba, bb = bufs[slot][:2]

        def addrow(r, carry):
            for k in range(D // 16):
                sl = pl.ds(k * 16, 16)
                plsc.addupdate(ba.at[r, sl], bb[r, sl])
            return carry

        lax.fori_loop(0, CH, addrow, 0)
        pltpu.sync_copy(ba, g.at[pl.ds(base + j * CH, CH)])

    _pipelined_chunks(nch, start, lambda: wait(0), lambda: wait(1), finish)


def _gather(row3d, col3d, xa, xb, n_edges):
    ept = n_edges // NW
    nch = ept // CH
    mesh = plsc.VectorSubcoreMesh(core_axis_name="c", subcore_axis_name="s")
    f = pl.kernel(
        functools.partial(_gather_body, ept, nch),
        out_type=jax.ShapeDtypeStruct((n_edges, D), jnp.float32),
        mesh=mesh,
        scratch_types=[
            pltpu.VMEM((nch, CH), jnp.int32),
            pltpu.VMEM((nch, CH), jnp.int32),
            pltpu.VMEM((CH, D), jnp.float32),
            pltpu.VMEM((CH, D), jnp.float32),
            pltpu.VMEM((CH, D), jnp.float32),
            pltpu.VMEM((CH, D), jnp.float32),
            pltpu.SemaphoreType.DMA,
            pltpu.SemaphoreType.DMA,
            pltpu.SemaphoreType.DMA,
            pltpu.SemaphoreType.DMA,
        ],
        compiler_params=pltpu.CompilerParams(use_tc_tiling_on_sc=False),
    )
    return f(row3d, col3d, xa, xb)


BE = 1280


def _mlp_body(g_ref, w2_ref, b2_ref, h_ref):
    g = jax.nn.softplus(g_ref[...])
    z = jnp.dot(g, w2_ref[...], preferred_element_type=jnp.float32) + b2_ref[...]
    h_ref[...] = jax.nn.softplus(z)


def _mlp(g, w2, b2):
    n_edges = g.shape[0]
    return pl.pallas_call(
        _mlp_body,
        grid=(n_edges // BE,),
        in_specs=[
            pl.BlockSpec((BE, D), lambda i: (i, 0)),
            pl.BlockSpec((D, D), lambda i: (0, 0)),
            pl.BlockSpec((1, D), lambda i: (0, 0)),
        ],
        out_specs=pl.BlockSpec((BE, D), lambda i: (i, 0)),
        out_shape=jax.ShapeDtypeStruct((n_edges, D), jnp.float32),
    )(g, w2, b2)


def _scatter_body(ept, nch, row3d, h, parts, cnts,
                  ir, hb0, hb1, ones, acc, acc_c, sh0, sh1):
    c = lax.axis_index("c")
    s = lax.axis_index("s")
    wid = s * NC + c
    base = wid * ept

    zv = jnp.zeros((16,), jnp.float32)
    ov = jnp.ones((16,), jnp.float32)

    def fill_z(r, carry):
        for k in range(D // 16):
            hb0[r, pl.ds(k * 16, 16)] = zv
        ones[r, :] = zv
        return carry

    lax.fori_loop(0, CH, fill_z, 0)

    def zero_stripe(t, carry):
        off = s * ROWS_PER_TILE + t * CH
        pltpu.sync_copy(hb0, acc.at[pl.ds(off, CH)])
        pltpu.sync_copy(ones, acc_c.at[pl.ds(off, CH)])
        return carry

    lax.fori_loop(0, ROWS_PER_TILE // CH, zero_stripe, 0)

    def fill_ones(r, carry):
        ones[r, :] = ov
        return carry

    lax.fori_loop(0, CH, fill_ones, 0)
    pltpu.sync_copy(row3d.at[wid], ir)
    plsc.subcore_barrier()

    bufs = ((hb0, sh0), (hb1, sh1))

    def start(j, slot):
        hb, sh = bufs[slot]
        pltpu.async_copy(h.at[pl.ds(base + j * CH, CH)], hb, sh)

    def wait(slot):
        hb, sh = bufs[slot]
        pltpu.make_async_copy(h.at[pl.ds(0, CH)], hb, sh).wait()

    def finish(j, slot):
        hb = bufs[slot][0]
        pltpu.sync_copy(hb, acc.at[ir.at[j]], add=True)
        pltpu.sync_copy(ones, acc_c.at[ir.at[j]], add=True)

    _pipelined_chunks(nch, start, lambda: wait(0), lambda: wait(1), finish)
    plsc.subcore_barrier()

    stripe = pl.ds(s * ROWS_PER_TILE, ROWS_PER_TILE)
    pltpu.sync_copy(acc.at[stripe], parts.at[c].at[stripe])
    pltpu.sync_copy(acc_c.at[stripe], cnts.at[c].at[stripe])


def _scatter(row3d, h):
    n_edges = h.shape[0]
    ept = n_edges // NW
    nch = ept // CH
    mesh = plsc.VectorSubcoreMesh(core_axis_name="c", subcore_axis_name="s")
    f = pl.kernel(
        functools.partial(_scatter_body, ept, nch),
        out_type=[
            jax.ShapeDtypeStruct((NC, N_PAD, D), jnp.float32),
            jax.ShapeDtypeStruct((NC, N_PAD, 16), jnp.float32),
        ],
        mesh=mesh,
        scratch_types=[
            pltpu.VMEM((nch, CH), jnp.int32),
            pltpu.VMEM((CH, D), jnp.float32),
            pltpu.VMEM((CH, D), jnp.float32),
            pltpu.VMEM((CH, 16), jnp.float32),
            pltpu.VMEM_SHARED((N_PAD, D), jnp.float32),
            pltpu.VMEM_SHARED((N_PAD, 16), jnp.float32),
            pltpu.SemaphoreType.DMA,
            pltpu.SemaphoreType.DMA,
        ],
        compiler_params=pltpu.CompilerParams(use_tc_tiling_on_sc=False),
    )
    return f(row3d, h)


NP_TOT = 2 * NB
BN = 1280


def _final_body(*refs):
    parts = refs[:NB]
    cnts = refs[NB:2 * NB]
    w3_ref, b3_ref, out_ref = refs[2 * NB:]
    p = parts[0][0] + parts[0][1]
    cnt = cnts[0][0, :, 0:1] + cnts[0][1, :, 0:1]
    for i in range(1, NB):
        p = p + parts[i][0] + parts[i][1]
        cnt = cnt + cnts[i][0, :, 0:1] + cnts[i][1, :, 0:1]
    out_ref[...] = (
        jnp.dot(p, w3_ref[...], preferred_element_type=jnp.float32)
        + cnt * b3_ref[...]
    )


def _final(parts, cnts, w3, b3):
    return pl.pallas_call(
        _final_body,
        grid=(N_PAD // BN,),
        in_specs=(
            [pl.BlockSpec((NC, BN, D), lambda i: (0, i, 0))] * NB
            + [pl.BlockSpec((NC, BN, 16), lambda i: (0, i, 0))] * NB
            + [
                pl.BlockSpec((D, D), lambda i: (0, 0)),
                pl.BlockSpec((1, D), lambda i: (0, 0)),
            ]
        ),
        out_specs=pl.BlockSpec((BN, D), lambda i: (i, 0)),
        out_shape=jax.ShapeDtypeStruct((N_PAD, D), jnp.float32),
    )(*parts, *cnts, w3, b3)


def kernel(x, edge_idx, W1, b1, W2, b2, W3, b3):
    row = edge_idx[0].astype(jnp.int32)
    col = edge_idx[1].astype(jnp.int32)
    xa, xb = _precompute(x, W1[:D], W1[D:], b1.reshape(1, D))

    ept = EB // NW
    parts, cnts = [], []
    for k in range(NB):
        sl = slice(k * EB, (k + 1) * EB)
        r3 = row[sl].reshape(NW, ept // CH, CH)
        c3 = col[sl].reshape(NW, ept // CH, CH)
        g = _gather(r3, c3, xa, xb, EB)
        h = _mlp(g, W2, b2.reshape(1, D))
        p, ct = _scatter(r3, h)
        parts.append(p)
        cnts.append(ct)

    return _final(parts, cnts, W3, b3.reshape(1, D))[:N_NODES]
